# Initial kernel scaffold; baseline (speedup 1.0000x reference)
import jax
import jax.numpy as jnp
from jax.experimental import pallas as pl


def kernel(x, W_base, W_other, select_mask):
    raise NotImplementedError("write your pallas kernel here")


# speedup vs baseline: 20.8172x; 20.8172x over previous
import functools

import jax
import jax.numpy as jnp
from jax import lax
from jax.experimental import pallas as pl
from jax.experimental.pallas import tpu as pltpu
from jax.experimental.pallas import tpu_sc as plsc

VOCAB = 100000
DIM = 64
PAD = 0

B = 4096
L = 200
NTOK = B * L


_VB = 2000
_XB = NTOK // 128 // (VOCAB // _VB)


def _prep_body(wb_ref, wo_ref, sel_ref, x_ref, merged_ref, bm_ref):
    sel = sel_ref[...]
    merged_ref[...] = jnp.where(sel != 0, wo_ref[...], wb_ref[...])
    bm_ref[...] = (x_ref[...] != PAD).astype(jnp.int32)


def _prep(W_base, W_other, sel2, xr):
    nsteps = VOCAB // _VB
    return pl.pallas_call(
        _prep_body,
        grid=(nsteps,),
        in_specs=[
            pl.BlockSpec((_VB, DIM), lambda i: (i, 0)),
            pl.BlockSpec((_VB, DIM), lambda i: (i, 0)),
            pl.BlockSpec((_VB, 1), lambda i: (i, 0)),
            pl.BlockSpec((_XB, 128), lambda i: (i, 0)),
        ],
        out_specs=[
            pl.BlockSpec((_VB, DIM), lambda i: (i, 0)),
            pl.BlockSpec((_XB, 128), lambda i: (i, 0)),
        ],
        out_shape=[
            jax.ShapeDtypeStruct((VOCAB, DIM), jnp.float32),
            jax.ShapeDtypeStruct((NTOK // 128, 128), jnp.int32),
        ],
    )(W_base, W_other, sel2, xr)


_SUB = 128


_NBUF = 4


def _make_gather():
    info = plsc.get_sparse_core_info()
    nw = info.num_cores * info.num_subcores
    per_w = NTOK // nw
    nsub = per_w // _SUB
    mesh = plsc.VectorSubcoreMesh(core_axis_name="c", subcore_axis_name="s")

    @functools.partial(
        pl.kernel,
        mesh=mesh,
        compiler_params=pltpu.CompilerParams(use_tc_tiling_on_sc=False),
        out_type=jax.ShapeDtypeStruct((NTOK, DIM), jnp.float32),
        scratch_types=[
            pltpu.VMEM((nsub, _SUB), jnp.int32),
            pltpu.VMEM((_NBUF, _SUB, DIM), jnp.float32),
            pltpu.SemaphoreType.DMA,
            pltpu.SemaphoreType.DMA,
        ],
    )
    def gather_k(merged_hbm, xr_hbm, out_hbm, idx_v, rows_v, gsem, wsem):
        c = lax.axis_index("c")
        s = lax.axis_index("s")
        wid = s * info.num_cores + c
        base = wid * per_w

        pltpu.sync_copy(xr_hbm.at[pl.ds(wid * nsub, nsub)], idx_v)
        pltpu.async_copy(merged_hbm.at[idx_v.at[0]], rows_v.at[0], gsem)

        def step(j, b):
            bn = (b + 1) % _NBUF
            pltpu.make_async_copy(
                merged_hbm.at[idx_v.at[0]], rows_v.at[b], gsem).wait()

            @pl.when(j >= _NBUF - 1)
            def _():
                pltpu.make_async_copy(
                    rows_v.at[bn], out_hbm.at[pl.ds(base, _SUB)], wsem).wait()

            @pl.when(j + 1 < nsub)
            def _():
                pltpu.async_copy(
                    merged_hbm.at[idx_v.at[j + 1]], rows_v.at[bn], gsem)

            pltpu.async_copy(
                rows_v.at[b], out_hbm.at[pl.ds(base + j * _SUB, _SUB)], wsem)

        def group(g, carry):
            for b in range(_NBUF):
                step(g * _NBUF + b, b)
            return carry

        lax.fori_loop(0, nsub // _NBUF, group, 0)

        for b in range(_NBUF - 1):
            pltpu.make_async_copy(
                rows_v.at[b], out_hbm.at[pl.ds(base, _SUB)], wsem).wait()

    return gather_k


_gather = _make_gather()


def kernel(x, W_base, W_other, select_mask):
    xr = x.reshape(NTOK // 128, 128)
    sel2 = select_mask.reshape(VOCAB, 1)
    merged, bm = _prep(W_base, W_other, sel2, xr)
    outflat = _gather(merged, xr)
    ret = outflat.reshape(B, L, DIM)
    basemask = bm.reshape(B, L)
    return (ret, basemask)


# speedup vs baseline: 25.9188x; 1.2451x over previous
import functools

import jax
import jax.numpy as jnp
from jax import lax
from jax.experimental import pallas as pl
from jax.experimental.pallas import tpu as pltpu
from jax.experimental.pallas import tpu_sc as plsc

VOCAB = 100000
DIM = 64
PAD = 0

B = 4096
L = 200
NTOK = B * L


_VB = 2000


def _merge_body(wb_ref, wo_ref, sel_ref, merged_ref):
    sel = sel_ref[...]
    merged_ref[:, :DIM] = jnp.where(sel != 0, wo_ref[...], wb_ref[...])
    merged_ref[:, DIM:] = jnp.zeros((_VB, DIM), jnp.float32)


def _prep_merged(W_base, W_other, sel2):
    return pl.pallas_call(
        _merge_body,
        grid=(VOCAB // _VB,),
        in_specs=[
            pl.BlockSpec((_VB, DIM), lambda i: (i, 0)),
            pl.BlockSpec((_VB, DIM), lambda i: (i, 0)),
            pl.BlockSpec((_VB, 1), lambda i: (i, 0)),
        ],
        out_specs=pl.BlockSpec((_VB, 2 * DIM), lambda i: (i, 0)),
        out_shape=jax.ShapeDtypeStruct((VOCAB, 2 * DIM), jnp.float32),
    )(W_base, W_other, sel2)


_XB = 128


def _bm_body(x_ref, bm_ref):
    bm_ref[...] = (x_ref[...] != PAD).astype(jnp.int32)


def _prep_bm(x):
    return pl.pallas_call(
        _bm_body,
        grid=(B // _XB,),
        in_specs=[pl.BlockSpec((_XB, L), lambda i: (i, 0))],
        out_specs=pl.BlockSpec((_XB, L), lambda i: (i, 0)),
        out_shape=jax.ShapeDtypeStruct((B, L), jnp.int32),
    )(x)


_LA = 96
_LB = L - _LA


def _make_gather():
    info = plsc.get_sparse_core_info()
    nw = info.num_cores * info.num_subcores
    bpw = B // nw
    half = bpw // 2
    npair = half // 2
    mesh = plsc.VectorSubcoreMesh(core_axis_name="c", subcore_axis_name="s")

    @functools.partial(
        pl.kernel,
        mesh=mesh,
        out_type=jax.ShapeDtypeStruct((B, L, DIM), jnp.float32),
        scratch_types=[
            pltpu.VMEM((half * L,), jnp.int32),
            pltpu.VMEM((2, _LA, 2 * DIM), jnp.float32),
            pltpu.VMEM((2, _LB, 2 * DIM), jnp.float32),
            pltpu.VMEM((2, _LA, DIM), jnp.float32),
            pltpu.VMEM((2, _LB, DIM), jnp.float32),
            pltpu.SemaphoreType.DMA,
            pltpu.SemaphoreType.DMA,
        ],
    )
    def gather_k(merged_hbm, x_hbm, out_hbm, xbuf, rowsa, rowsb, wba, wbb,
                 gsem, wsem):
        c = lax.axis_index("c")
        s = lax.axis_index("s")
        wid = s * info.num_cores + c
        b0 = wid * bpw

        def fire_a(j):
            for r in range(2):
                pltpu.async_copy(
                    merged_hbm.at[xbuf.at[pl.ds((2 * j + r) * L, _LA)]],
                    rowsa.at[r], gsem)

        def fire_b(j):
            for r in range(2):
                pltpu.async_copy(
                    merged_hbm.at[xbuf.at[pl.ds((2 * j + r) * L + _LA, _LB)]],
                    rowsb.at[r], gsem)

        def wait_g(rows, n):
            for r in range(2):
                pltpu.make_async_copy(
                    merged_hbm.at[xbuf.at[pl.ds(0, n)]],
                    rows.at[r], gsem).wait()

        def wait_w_a():
            pltpu.make_async_copy(
                wba, out_hbm.at[pl.ds(b0, 2), pl.ds(0, _LA)], wsem).wait()

        def wait_w_b():
            pltpu.make_async_copy(
                wbb, out_hbm.at[pl.ds(b0, 2), pl.ds(_LA, _LB)], wsem).wait()

        def extract(wb, rows, n):
            def body(j, carry):
                for r in range(2):
                    for k in range(DIM // 16):
                        wb[r, j, pl.ds(k * 16, 16)] = (
                            rows[r, j, pl.ds(k * 16, 16)])
                return carry

            lax.fori_loop(0, n, body, 0)

        for h in range(2):
            hb = b0 + h * half
            pltpu.sync_copy(x_hbm.at[pl.ds(hb * L, half * L)], xbuf)
            fire_a(0)
            fire_b(0)

            def pair_body(j, carry):
                bcur = hb + 2 * j

                wait_g(rowsa, _LA)

                @pl.when(j >= 1)
                def _():
                    wait_w_a()

                extract(wba, rowsa, _LA)
                pltpu.async_copy(
                    wba, out_hbm.at[pl.ds(bcur, 2), pl.ds(0, _LA)], wsem)

                @pl.when(j + 1 < npair)
                def _():
                    fire_a(j + 1)

                wait_g(rowsb, _LB)

                @pl.when(j >= 1)
                def _():
                    wait_w_b()

                extract(wbb, rowsb, _LB)
                pltpu.async_copy(
                    wbb, out_hbm.at[pl.ds(bcur, 2), pl.ds(_LA, _LB)], wsem)

                @pl.when(j + 1 < npair)
                def _():
                    fire_b(j + 1)

                return carry

            lax.fori_loop(0, npair, pair_body, 0)
            wait_w_a()
            wait_w_b()

    return gather_k


_gather = _make_gather()


def kernel(x, W_base, W_other, select_mask):
    sel2 = select_mask.reshape(VOCAB, 1)
    merged = _prep_merged(W_base, W_other, sel2)
    ret = _gather(merged, x.reshape(NTOK))
    basemask = _prep_bm(x)
    return (ret, basemask)


# speedup vs baseline: 26.3229x; 1.0156x over previous
import functools

import jax
import jax.numpy as jnp
from jax import lax
from jax.experimental import pallas as pl
from jax.experimental.pallas import tpu as pltpu
from jax.experimental.pallas import tpu_sc as plsc

VOCAB = 100000
DIM = 64
PAD = 0

B = 4096
L = 200
NTOK = B * L


_VB = 4000


def _merge_body(wb_ref, wo_ref, sel_ref, merged_ref):
    sel = sel_ref[...]
    merged_ref[:, :DIM] = jnp.where(sel != 0, wo_ref[...], wb_ref[...])
    merged_ref[:, DIM:] = jnp.zeros((_VB, DIM), jnp.float32)


def _prep_merged(W_base, W_other, sel2):
    return pl.pallas_call(
        _merge_body,
        grid=(VOCAB // _VB,),
        in_specs=[
            pl.BlockSpec((_VB, DIM), lambda i: (i, 0)),
            pl.BlockSpec((_VB, DIM), lambda i: (i, 0)),
            pl.BlockSpec((_VB, 1), lambda i: (i, 0)),
        ],
        out_specs=pl.BlockSpec((_VB, 2 * DIM), lambda i: (i, 0)),
        out_shape=jax.ShapeDtypeStruct((VOCAB, 2 * DIM), jnp.float32),
    )(W_base, W_other, sel2)


_LBK = 8


def _bm_body(xt_ref, bmt_ref):
    bmt_ref[...] = (xt_ref[...] != PAD).astype(jnp.int32)


def _prep_bm(xT):
    return pl.pallas_call(
        _bm_body,
        grid=(L // _LBK,),
        in_specs=[pl.BlockSpec((_LBK, B), lambda i: (i, 0))],
        out_specs=pl.BlockSpec((_LBK, B), lambda i: (i, 0)),
        out_shape=jax.ShapeDtypeStruct((L, B), jnp.int32),
    )(xT)


_LA = 96
_LB = L - _LA


def _make_gather():
    info = plsc.get_sparse_core_info()
    nw = info.num_cores * info.num_subcores
    bpw = B // nw
    half = bpw // 2
    npair = half // 2
    mesh = plsc.VectorSubcoreMesh(core_axis_name="c", subcore_axis_name="s")

    @functools.partial(
        pl.kernel,
        mesh=mesh,
        out_type=jax.ShapeDtypeStruct((B, L, DIM), jnp.float32),
        scratch_types=[
            pltpu.VMEM((half * L,), jnp.int32),
            pltpu.VMEM((2, _LA, 2 * DIM), jnp.float32),
            pltpu.VMEM((2, _LB, 2 * DIM), jnp.float32),
            pltpu.VMEM((2, _LA, DIM), jnp.float32),
            pltpu.VMEM((2, _LB, DIM), jnp.float32),
            pltpu.SemaphoreType.DMA,
            pltpu.SemaphoreType.DMA,
        ],
    )
    def gather_k(merged_hbm, x_hbm, out_hbm, xbuf, rowsa, rowsb, wba, wbb,
                 gsem, wsem):
        c = lax.axis_index("c")
        s = lax.axis_index("s")
        wid = s * info.num_cores + c
        b0 = wid * bpw

        def fire_a(j):
            for r in range(2):
                pltpu.async_copy(
                    merged_hbm.at[xbuf.at[pl.ds((2 * j + r) * L, _LA)]],
                    rowsa.at[r], gsem)

        def fire_b(j):
            for r in range(2):
                pltpu.async_copy(
                    merged_hbm.at[xbuf.at[pl.ds((2 * j + r) * L + _LA, _LB)]],
                    rowsb.at[r], gsem)

        def wait_g(rows, n):
            for r in range(2):
                pltpu.make_async_copy(
                    merged_hbm.at[xbuf.at[pl.ds(0, n)]],
                    rows.at[r], gsem).wait()

        def wait_w_a():
            pltpu.make_async_copy(
                wba, out_hbm.at[pl.ds(b0, 2), pl.ds(0, _LA)], wsem).wait()

        def wait_w_b():
            pltpu.make_async_copy(
                wbb, out_hbm.at[pl.ds(b0, 2), pl.ds(_LA, _LB)], wsem).wait()

        def extract(wb, rows, n):
            def body(j, carry):
                for r in range(2):
                    for k in range(DIM // 16):
                        wb[r, j, pl.ds(k * 16, 16)] = (
                            rows[r, j, pl.ds(k * 16, 16)])
                return carry

            lax.fori_loop(0, n, body, 0)

        for h in range(2):
            hb = b0 + h * half
            pltpu.sync_copy(x_hbm.at[pl.ds(hb * L, half * L)], xbuf)
            fire_a(0)
            fire_b(0)

            def pair_body(j, carry):
                bcur = hb + 2 * j

                wait_g(rowsa, _LA)

                @pl.when(j >= 1)
                def _():
                    wait_w_a()

                extract(wba, rowsa, _LA)
                pltpu.async_copy(
                    wba, out_hbm.at[pl.ds(bcur, 2), pl.ds(0, _LA)], wsem)

                @pl.when(j + 1 < npair)
                def _():
                    fire_a(j + 1)

                wait_g(rowsb, _LB)

                @pl.when(j >= 1)
                def _():
                    wait_w_b()

                extract(wbb, rowsb, _LB)
                pltpu.async_copy(
                    wbb, out_hbm.at[pl.ds(bcur, 2), pl.ds(_LA, _LB)], wsem)

                @pl.when(j + 1 < npair)
                def _():
                    fire_b(j + 1)

                return carry

            lax.fori_loop(0, npair, pair_body, 0)
            wait_w_a()
            wait_w_b()

    return gather_k


_gather = _make_gather()


def kernel(x, W_base, W_other, select_mask):
    sel2 = select_mask.astype(jnp.int8).reshape(VOCAB, 1)
    merged = _prep_merged(W_base, W_other, sel2)
    ret = _gather(merged, x.reshape(NTOK))
    basemask = _prep_bm(x.T).T
    return (ret, basemask)


# speedup vs baseline: 27.4192x; 1.0416x over previous
import functools

import jax
import jax.numpy as jnp
from jax import lax
from jax.experimental import pallas as pl
from jax.experimental.pallas import tpu as pltpu
from jax.experimental.pallas import tpu_sc as plsc

VOCAB = 100000
DIM = 64
PAD = 0

B = 4096
L = 200
NTOK = B * L


_VB = 4000


def _merge_body(wb_ref, wo_ref, sel_ref, merged_ref):
    sel = sel_ref[...]
    merged_ref[:, :DIM] = jnp.where(sel != 0, wo_ref[...], wb_ref[...])
    merged_ref[:, DIM:] = jnp.zeros((_VB, DIM), jnp.float32)


def _prep_merged(W_base, W_other, sel2):
    return pl.pallas_call(
        _merge_body,
        grid=(VOCAB // _VB,),
        in_specs=[
            pl.BlockSpec((_VB, DIM), lambda i: (i, 0)),
            pl.BlockSpec((_VB, DIM), lambda i: (i, 0)),
            pl.BlockSpec((_VB, 1), lambda i: (i, 0)),
        ],
        out_specs=pl.BlockSpec((_VB, 2 * DIM), lambda i: (i, 0)),
        out_shape=jax.ShapeDtypeStruct((VOCAB, 2 * DIM), jnp.float32),
    )(W_base, W_other, sel2)


_LBK = 8


def _bm_body(xt_ref, bmt_ref):
    bmt_ref[...] = (xt_ref[...] != PAD).astype(jnp.int32)


def _prep_bm(xT):
    return pl.pallas_call(
        _bm_body,
        grid=(L // _LBK,),
        in_specs=[pl.BlockSpec((_LBK, B), lambda i: (i, 0))],
        out_specs=pl.BlockSpec((_LBK, B), lambda i: (i, 0)),
        out_shape=jax.ShapeDtypeStruct((L, B), jnp.int32),
    )(xT)


_TLB = 8
_TBB = 2048


def _tr_body(in_ref, out_ref):
    for l in range(_TLB):
        out_ref[l] = in_ref[:, l, :].T


def _transpose_out(ret_rm):
    return pl.pallas_call(
        _tr_body,
        grid=(L // _TLB, B // _TBB),
        in_specs=[pl.BlockSpec((_TBB, _TLB, DIM), lambda i, j: (j, i, 0))],
        out_specs=pl.BlockSpec((_TLB, DIM, _TBB), lambda i, j: (i, 0, j)),
        out_shape=jax.ShapeDtypeStruct((L, DIM, B), jnp.float32),
    )(ret_rm)


_LA = 96
_LB = L - _LA


def _make_gather():
    info = plsc.get_sparse_core_info()
    nw = info.num_cores * info.num_subcores
    bpw = B // nw
    half = bpw // 2
    npair = half // 2
    mesh = plsc.VectorSubcoreMesh(core_axis_name="c", subcore_axis_name="s")

    @functools.partial(
        pl.kernel,
        mesh=mesh,
        out_type=jax.ShapeDtypeStruct((B, L, DIM), jnp.float32),
        scratch_types=[
            pltpu.VMEM((half * L,), jnp.int32),
            pltpu.VMEM((2, _LA, 2 * DIM), jnp.float32),
            pltpu.VMEM((2, _LB, 2 * DIM), jnp.float32),
            pltpu.VMEM((2, _LA, DIM), jnp.float32),
            pltpu.VMEM((2, _LB, DIM), jnp.float32),
            pltpu.SemaphoreType.DMA,
            pltpu.SemaphoreType.DMA,
        ],
    )
    def gather_k(merged_hbm, x_hbm, out_hbm, xbuf, rowsa, rowsb, wba, wbb,
                 gsem, wsem):
        c = lax.axis_index("c")
        s = lax.axis_index("s")
        wid = s * info.num_cores + c
        b0 = wid * bpw

        def fire_a(j):
            for r in range(2):
                pltpu.async_copy(
                    merged_hbm.at[xbuf.at[pl.ds((2 * j + r) * L, _LA)]],
                    rowsa.at[r], gsem)

        def fire_b(j):
            for r in range(2):
                pltpu.async_copy(
                    merged_hbm.at[xbuf.at[pl.ds((2 * j + r) * L + _LA, _LB)]],
                    rowsb.at[r], gsem)

        def wait_g(rows, n):
            for r in range(2):
                pltpu.make_async_copy(
                    merged_hbm.at[xbuf.at[pl.ds(0, n)]],
                    rows.at[r], gsem).wait()

        def wait_w_a():
            pltpu.make_async_copy(
                wba, out_hbm.at[pl.ds(b0, 2), pl.ds(0, _LA)], wsem).wait()

        def wait_w_b():
            pltpu.make_async_copy(
                wbb, out_hbm.at[pl.ds(b0, 2), pl.ds(_LA, _LB)], wsem).wait()

        def extract(wb, rows, n):
            def body(j, carry):
                for r in range(2):
                    for k in range(DIM // 16):
                        wb[r, j, pl.ds(k * 16, 16)] = (
                            rows[r, j, pl.ds(k * 16, 16)])
                return carry

            lax.fori_loop(0, n, body, 0)

        for h in range(2):
            hb = b0 + h * half
            pltpu.sync_copy(x_hbm.at[pl.ds(hb * L, half * L)], xbuf)
            fire_a(0)
            fire_b(0)

            def pair_body(j, carry):
                bcur = hb + 2 * j

                wait_g(rowsa, _LA)

                @pl.when(j >= 1)
                def _():
                    wait_w_a()

                extract(wba, rowsa, _LA)
                pltpu.async_copy(
                    wba, out_hbm.at[pl.ds(bcur, 2), pl.ds(0, _LA)], wsem)

                @pl.when(j + 1 < npair)
                def _():
                    fire_a(j + 1)

                wait_g(rowsb, _LB)

                @pl.when(j >= 1)
                def _():
                    wait_w_b()

                extract(wbb, rowsb, _LB)
                pltpu.async_copy(
                    wbb, out_hbm.at[pl.ds(bcur, 2), pl.ds(_LA, _LB)], wsem)

                @pl.when(j + 1 < npair)
                def _():
                    fire_b(j + 1)

                return carry

            lax.fori_loop(0, npair, pair_body, 0)
            wait_w_a()
            wait_w_b()

    return gather_k


_gather = _make_gather()


def kernel(x, W_base, W_other, select_mask):
    sel2 = select_mask.astype(jnp.int8).reshape(VOCAB, 1)
    merged = _prep_merged(W_base, W_other, sel2)
    ret = _transpose_out(_gather(merged, x.reshape(NTOK))).transpose(2, 0, 1)
    basemask = _prep_bm(x.T).T
    return (ret, basemask)
